# baseline (speedup 1.0000x reference)
import functools
import numpy as np
import jax
import jax.numpy as jnp
from jax import lax
from jax.experimental import pallas as pl
from jax.experimental.pallas import tpu as pltpu
from jax.experimental.pallas import tpu_sc as plsc

N = 10000
E = 160000
D = 256
H = 128
NA = 21
ORDER = 10
MAXEL = 94

NSUB = 16
CH = 64
NCH = 160
EP = NSUB * CH * NCH
NP = 10240
STRIPE = NP // NSUB

_INV_SQRT_PI = float(1.0 / np.sqrt(np.pi))
_CONST0 = float(1.0 / np.sqrt(2.0) / np.sqrt(np.pi))


def _prep_body(an_ref, embf_ref, embh_ref, w1_ref, w2_ref, a1_ref, a2_ref, af_ref):
    an = an_ref[...]
    oh = (lax.broadcasted_iota(jnp.int32, (an.shape[0], MAXEL), 1) == an)
    oh = oh.astype(jnp.float32)
    emb = embf_ref[...]
    t1 = jnp.dot(emb, w1_ref[...], preferred_element_type=jnp.float32)
    t2 = jnp.dot(emb, w2_ref[...], preferred_element_type=jnp.float32)
    a1_ref[...] = jnp.dot(oh, t1, preferred_element_type=jnp.float32)
    a2_ref[...] = jnp.dot(oh, t2, preferred_element_type=jnp.float32)
    af_ref[...] = jnp.dot(oh, embh_ref[...], preferred_element_type=jnp.float32)


def _prep(an2d, embedding, w1, w2):
    BA = 2000
    grid = (N // BA, 2)
    return pl.pallas_call(
        _prep_body,
        grid=grid,
        in_specs=[
            pl.BlockSpec((BA, 1), lambda i, h: (i, 0)),
            pl.BlockSpec((MAXEL, D), lambda i, h: (0, 0)),
            pl.BlockSpec((MAXEL, H), lambda i, h: (0, h)),
            pl.BlockSpec((D, H), lambda i, h: (0, h)),
            pl.BlockSpec((D, H), lambda i, h: (0, h)),
        ],
        out_specs=[
            pl.BlockSpec((BA, H), lambda i, h: (h * (N // BA) + i, 0)),
            pl.BlockSpec((BA, H), lambda i, h: (h * (N // BA) + i, 0)),
            pl.BlockSpec((BA, H), lambda i, h: (i, h)),
        ],
        out_shape=[
            jax.ShapeDtypeStruct((2 * N, H), jnp.float32),
            jax.ShapeDtypeStruct((2 * N, H), jnp.float32),
            jax.ShapeDtypeStruct((N, D), jnp.float32),
        ],
    )(an2d, embedding, embedding, w1, w2)


def _angle_body(bi_ref, bj_ref, wang_ref, w3_ref, p_ref, fea_ref):
    h = pl.program_id(1)

    @pl.when(h == 0)
    def _():
        bi = bi_ref[...]
        bj = bj_ref[...]
        ni = jnp.sqrt(jnp.sum(bi * bi, axis=1, keepdims=True)) + 1e-12
        nj = jnp.sqrt(jnp.sum(bj * bj, axis=1, keepdims=True)) + 1e-12
        c = jnp.sum(bi * bj, axis=1, keepdims=True) / (ni * nj) * (1.0 - 1e-6)
        s = jnp.sqrt(jnp.maximum(0.0, 1.0 - c * c))
        fea_ref[:, 0:1] = jnp.full_like(c, _CONST0)
        ckm1 = jnp.ones_like(c)
        ck = c
        skm1 = jnp.zeros_like(c)
        sk = s
        for n in range(1, ORDER + 1):
            fea_ref[:, n:n + 1] = ck * _INV_SQRT_PI
            fea_ref[:, ORDER + n:ORDER + n + 1] = sk * _INV_SQRT_PI
            ckp = 2.0 * c * ck - ckm1
            skp = 2.0 * c * sk - skm1
            ckm1, ck = ck, ckp
            skm1, sk = sk, skp

    wa = jnp.dot(wang_ref[...], w3_ref[...], preferred_element_type=jnp.float32)
    p_ref[...] = jnp.dot(fea_ref[...], wa, preferred_element_type=jnp.float32)


def _angle(bi_p, bj_p, w_angle, w3):
    BE = 2048
    grid = (EP // BE, 2)
    return pl.pallas_call(
        _angle_body,
        grid=grid,
        in_specs=[
            pl.BlockSpec((BE, 3), lambda i, h: (i, 0)),
            pl.BlockSpec((BE, 3), lambda i, h: (i, 0)),
            pl.BlockSpec((NA, D), lambda i, h: (0, 0)),
            pl.BlockSpec((D, H), lambda i, h: (0, h)),
        ],
        out_specs=pl.BlockSpec((BE, H), lambda i, h: (h * (EP // BE) + i, 0)),
        out_shape=jax.ShapeDtypeStruct((2 * EP, H), jnp.float32),
        scratch_shapes=[pltpu.VMEM((BE, NA), jnp.float32)],
    )(bi_p, bj_p, w_angle, w3)


def _sc_body(src_hbm, dst_hbm, a1_hbm, a2_hbm, p_hbm, b_hbm, out_hbm,
             srcv, dstm, idx1, idx2, r1, r2, pbuf, bbuf, zbuf, agg, sem):
    c = lax.axis_index("c")
    s = lax.axis_index("s")

    pltpu.sync_copy(b_hbm.at[pl.ds(c * H, H)], bbuf)

    def zrow(i, _):
        for k in range(H // 16):
            zbuf[i, pl.ds(k * 16, 16)] = jnp.zeros((16,), jnp.float32)
        return ()
    lax.fori_loop(0, CH, zrow, ())
    for k in range(STRIPE // CH):
        pltpu.sync_copy(zbuf, agg.at[pl.ds(s * STRIPE + k * CH, CH)])
    plsc.subcore_barrier()

    def chunk(j, _):
        eoff = s * (NCH * CH) + j * CH
        pltpu.sync_copy(src_hbm.at[pl.ds(eoff, CH)], srcv)
        pltpu.sync_copy(dst_hbm.at[pl.ds(eoff, CH)], dstm.at[0])
        base = c * N
        for k in range(CH // 16):
            idx1[pl.ds(k * 16, 16)] = srcv[pl.ds(k * 16, 16)] + base
            idx2[pl.ds(k * 16, 16)] = dstm[0, pl.ds(k * 16, 16)] + base
        pltpu.async_copy(a1_hbm.at[idx1], r1, sem).wait()
        pltpu.async_copy(a2_hbm.at[idx2], r2, sem).wait()
        pltpu.sync_copy(p_hbm.at[pl.ds(c * EP + eoff, CH)], pbuf)

        def row(i, _):
            for k in range(H // 16):
                sl = pl.ds(k * 16, 16)
                x = r1[i, sl] + r2[i, sl] + pbuf[i, sl] + bbuf[sl]
                r1[i, sl] = x / (1.0 + jnp.exp(-x))
            return ()
        lax.fori_loop(0, CH, row, ())

        pltpu.sync_copy(r1, agg.at[dstm.at[0]], add=True)
        return ()

    lax.fori_loop(0, NCH, chunk, ())
    plsc.subcore_barrier()

    for k in range(STRIPE // CH):
        roff = s * STRIPE + k * CH
        pltpu.sync_copy(agg.at[pl.ds(roff, CH)], zbuf)
        pltpu.sync_copy(zbuf, out_hbm.at[pl.ds(c * NP + roff, CH)])


def _sc_aggregate(src_p, dst_p, a1hh, a2hh, phh, b_msg):
    mesh = plsc.VectorSubcoreMesh(core_axis_name="c", subcore_axis_name="s")
    kern = functools.partial(
        pl.kernel,
        mesh=mesh,
        out_type=jax.ShapeDtypeStruct((2 * NP, H), jnp.float32),
        scratch_types=[
            pltpu.VMEM((CH,), jnp.int32),
            pltpu.VMEM((1, CH), jnp.int32),
            pltpu.VMEM((CH,), jnp.int32),
            pltpu.VMEM((CH,), jnp.int32),
            pltpu.VMEM((CH, H), jnp.float32),
            pltpu.VMEM((CH, H), jnp.float32),
            pltpu.VMEM((CH, H), jnp.float32),
            pltpu.VMEM((H,), jnp.float32),
            pltpu.VMEM((CH, H), jnp.float32),
            pltpu.VMEM_SHARED((NP, H), jnp.float32),
            pltpu.SemaphoreType.DMA,
        ],
    )(_sc_body)
    return kern(src_p, dst_p, a1hh, a2hh, phh, b_msg)


def _out_body(af_ref, aggl_ref, aggu_ref, wl_ref, wu_ref, o_ref):
    o_ref[...] = (af_ref[...]
                  + jnp.dot(aggl_ref[...], wl_ref[...], preferred_element_type=jnp.float32)
                  + jnp.dot(aggu_ref[...], wu_ref[...], preferred_element_type=jnp.float32))


def _outproj(af, aggl, aggu, w_out):
    BA = 2000
    return pl.pallas_call(
        _out_body,
        grid=(N // BA,),
        in_specs=[
            pl.BlockSpec((BA, D), lambda i: (i, 0)),
            pl.BlockSpec((BA, H), lambda i: (i, 0)),
            pl.BlockSpec((BA, H), lambda i: (i, 0)),
            pl.BlockSpec((H, D), lambda i: (0, 0)),
            pl.BlockSpec((H, D), lambda i: (0, 0)),
        ],
        out_specs=pl.BlockSpec((BA, D), lambda i: (i, 0)),
        out_shape=jax.ShapeDtypeStruct((N, D), jnp.float32),
    )(af, aggl, aggu, w_out[:H], w_out[H:])


def kernel(atomic_numbers, edge_index, bond_i, bond_j, embedding, W_angle, W_msg, b_msg, W_out):
    an2d = atomic_numbers.astype(jnp.int32).reshape(N, 1)
    w1 = W_msg[:D]
    w2 = W_msg[D:2 * D]
    w3 = W_msg[2 * D:]

    a1hh, a2hh, af = _prep(an2d, embedding, w1, w2)

    pad = EP - E
    bi_p = jnp.pad(bond_i, ((0, pad), (0, 0)))
    bj_p = jnp.pad(bond_j, ((0, pad), (0, 0)))
    phh = _angle(bi_p, bj_p, W_angle, w3)

    src_p = jnp.pad(edge_index[0].astype(jnp.int32), (0, pad))
    dst_p = jnp.pad(edge_index[1].astype(jnp.int32), (0, pad), constant_values=N)
    aggp = _sc_aggregate(src_p, dst_p, a1hh, a2hh, phh, b_msg)

    aggl = aggp[:N]
    aggu = aggp[NP:NP + N]
    return _outproj(af, aggl, aggu, W_out)


# speedup vs baseline: 2.6819x; 2.6819x over previous
import functools
import numpy as np
import jax
import jax.numpy as jnp
from jax import lax
from jax.experimental import pallas as pl
from jax.experimental.pallas import tpu as pltpu
from jax.experimental.pallas import tpu_sc as plsc

N = 10000
E = 160000
D = 256
H = 128
NA = 21
ORDER = 10
MAXEL = 94

NSUB = 16
CH = 64
NCH = 160
NG = NCH // 2
EP = NSUB * CH * NCH
NP = 10112
STRIPE = NP // NSUB

_INV_SQRT_PI = float(1.0 / np.sqrt(np.pi))
_CONST0 = float(1.0 / np.sqrt(2.0) / np.sqrt(np.pi))


def _prep_body(an_ref, embf_ref, embh_ref, w12_ref, t_ref, af_ref):
    an = an_ref[...]
    oh = (lax.broadcasted_iota(jnp.int32, (an.shape[0], MAXEL), 1) == an)
    oh = oh.astype(jnp.float32)
    tw = jnp.dot(embf_ref[...], w12_ref[...], preferred_element_type=jnp.float32)
    t_ref[...] = jnp.dot(oh, tw, preferred_element_type=jnp.float32)
    af_ref[...] = jnp.dot(oh, embh_ref[...], preferred_element_type=jnp.float32)


def _prep(an2d, embedding, w12):
    BA = 2000
    NB = N // BA
    grid = (NB, 2, 2)
    return pl.pallas_call(
        _prep_body,
        grid=grid,
        in_specs=[
            pl.BlockSpec((BA, 1), lambda i, h, t: (i, 0)),
            pl.BlockSpec((MAXEL, D), lambda i, h, t: (0, 0)),
            pl.BlockSpec((MAXEL, H), lambda i, h, t: (0, h)),
            pl.BlockSpec((D, H), lambda i, h, t: (t, h)),
        ],
        out_specs=[
            pl.BlockSpec((BA, H), lambda i, h, t: ((t * 2 + h) * NB + i, 0)),
            pl.BlockSpec((BA, H), lambda i, h, t: (i, h)),
        ],
        out_shape=[
            jax.ShapeDtypeStruct((4 * N, H), jnp.float32),
            jax.ShapeDtypeStruct((N, D), jnp.float32),
        ],
    )(an2d, embedding, embedding, w12)


def _angle_body(bi_ref, bj_ref, wang_ref, w3_ref, p_ref, fea_ref):
    h = pl.program_id(1)

    @pl.when(h == 0)
    def _():
        bi = bi_ref[...]
        bj = bj_ref[...]
        ni = jnp.sqrt(jnp.sum(bi * bi, axis=1, keepdims=True)) + 1e-12
        nj = jnp.sqrt(jnp.sum(bj * bj, axis=1, keepdims=True)) + 1e-12
        c = jnp.sum(bi * bj, axis=1, keepdims=True) / (ni * nj) * (1.0 - 1e-6)
        s = jnp.sqrt(jnp.maximum(0.0, 1.0 - c * c))
        fea_ref[:, 0:1] = jnp.full_like(c, _CONST0)
        ckm1 = jnp.ones_like(c)
        ck = c
        skm1 = jnp.zeros_like(c)
        sk = s
        for n in range(1, ORDER + 1):
            fea_ref[:, n:n + 1] = ck * _INV_SQRT_PI
            fea_ref[:, ORDER + n:ORDER + n + 1] = sk * _INV_SQRT_PI
            ckp = 2.0 * c * ck - ckm1
            skp = 2.0 * c * sk - skm1
            ckm1, ck = ck, ckp
            skm1, sk = sk, skp

    wa = jnp.dot(wang_ref[...], w3_ref[...], preferred_element_type=jnp.float32)
    p_ref[...] = jnp.dot(fea_ref[...], wa, preferred_element_type=jnp.float32)


def _angle(bi_p, bj_p, w_angle, w3):
    BE = 2048
    grid = (EP // BE, 2)
    return pl.pallas_call(
        _angle_body,
        grid=grid,
        in_specs=[
            pl.BlockSpec((BE, 3), lambda i, h: (i, 0)),
            pl.BlockSpec((BE, 3), lambda i, h: (i, 0)),
            pl.BlockSpec((NA, D), lambda i, h: (0, 0)),
            pl.BlockSpec((D, H), lambda i, h: (0, h)),
        ],
        out_specs=pl.BlockSpec((BE, H), lambda i, h: (h * (EP // BE) + i, 0)),
        out_shape=jax.ShapeDtypeStruct((2 * EP, H), jnp.float32),
        scratch_shapes=[pltpu.VMEM((BE, NA), jnp.float32)],
    )(bi_p, bj_p, w_angle, w3)


def _sc_body(sd_hbm, t_hbm, p_hbm, b_hbm, out_hbm,
             sdm0, sdm1, dstm0, dstm1, idx0, idx1, rc0, rc1, pb0, pb1, bbuf,
             agg, semA, semB):
    c = lax.axis_index("c")
    s = lax.axis_index("s")

    pltpu.sync_copy(b_hbm.at[pl.ds(c * H, H)], bbuf)

    @plsc.parallel_loop(0, 2 * CH)
    def _z(i):
        for k in range(H // 16):
            rc0[i, pl.ds(k * 16, 16)] = jnp.zeros((16,), jnp.float32)
    for k in range(4):
        pltpu.sync_copy(rc0, agg.at[pl.ds(s * STRIPE + k * 128, 128)])
    pltpu.sync_copy(rc0.at[pl.ds(0, STRIPE - 512)],
                    agg.at[pl.ds(s * STRIPE + 512, STRIPE - 512)])
    plsc.subcore_barrier()

    base1 = c * N
    base2 = 2 * N + c * N

    def prep(j, sdm, dstm, idx, rc, pb, sem):
        g = s * NCH + j
        pltpu.sync_copy(sd_hbm.at[pl.ds(g * 2 * CH, 2 * CH)], sdm.at[0])
        for k in range(CH // 16):
            sl = pl.ds(k * 16, 16)
            idx[sl] = sdm[0, sl] + base1
        for k in range(CH // 16):
            sli = pl.ds(CH + k * 16, 16)
            v = sdm[0, sli]
            idx[sli] = v + base2
            dstm[0, pl.ds(k * 16, 16)] = v
        pltpu.async_copy(t_hbm.at[idx], rc, sem)
        pltpu.async_copy(p_hbm.at[pl.ds(c * EP + g * CH, CH)], pb, sem)

    def drain(rc, pb, sem):
        pltpu.make_async_copy(t_hbm.at[idx0], rc, sem).wait()
        pltpu.make_async_copy(p_hbm.at[pl.ds(0, CH)], pb, sem).wait()

    def compute(rc, pb, dstm):
        @plsc.parallel_loop(0, CH)
        def _cmp(i):
            for k in range(H // 16):
                sl = pl.ds(k * 16, 16)
                x = rc[i, sl] + rc[CH + i, sl] + pb[i, sl] + bbuf[sl]
                rc[i, sl] = x / (1.0 + jnp.exp(-x))
        pltpu.sync_copy(rc.at[pl.ds(0, CH)], agg.at[dstm.at[0]], add=True)

    prep(0, sdm0, dstm0, idx0, rc0, pb0, semA)

    def gbody(g, _):
        j0 = 2 * g
        prep(j0 + 1, sdm1, dstm1, idx1, rc1, pb1, semB)
        drain(rc0, pb0, semA)
        compute(rc0, pb0, dstm0)

        @pl.when(g < NG - 1)
        def _():
            prep(j0 + 2, sdm0, dstm0, idx0, rc0, pb0, semA)
        drain(rc1, pb1, semB)
        compute(rc1, pb1, dstm1)
        return ()

    lax.fori_loop(0, NG, gbody, ())
    plsc.subcore_barrier()

    for k in range(4):
        pltpu.sync_copy(agg.at[pl.ds(s * STRIPE + k * 128, 128)], rc0)
        pltpu.sync_copy(rc0, out_hbm.at[pl.ds(c * NP + s * STRIPE + k * 128, 128)])
    pltpu.sync_copy(agg.at[pl.ds(s * STRIPE + 512, STRIPE - 512)],
                    rc0.at[pl.ds(0, STRIPE - 512)])
    pltpu.sync_copy(rc0.at[pl.ds(0, STRIPE - 512)],
                    out_hbm.at[pl.ds(c * NP + s * STRIPE + 512, STRIPE - 512)])


def _sc_aggregate(sd_p, table, phh, b_msg):
    mesh = plsc.VectorSubcoreMesh(core_axis_name="c", subcore_axis_name="s")
    kern = functools.partial(
        pl.kernel,
        mesh=mesh,
        out_type=jax.ShapeDtypeStruct((2 * NP, H), jnp.float32),
        scratch_types=[
            pltpu.VMEM((1, 2 * CH), jnp.int32),
            pltpu.VMEM((1, 2 * CH), jnp.int32),
            pltpu.VMEM((1, CH), jnp.int32),
            pltpu.VMEM((1, CH), jnp.int32),
            pltpu.VMEM((2 * CH,), jnp.int32),
            pltpu.VMEM((2 * CH,), jnp.int32),
            pltpu.VMEM((2 * CH, H), jnp.float32),
            pltpu.VMEM((2 * CH, H), jnp.float32),
            pltpu.VMEM((CH, H), jnp.float32),
            pltpu.VMEM((CH, H), jnp.float32),
            pltpu.VMEM((H,), jnp.float32),
            pltpu.VMEM_SHARED((NP, H), jnp.float32),
            pltpu.SemaphoreType.DMA,
            pltpu.SemaphoreType.DMA,
        ],
    )(_sc_body)
    return kern(sd_p, table, phh, b_msg)


def _out_body(af_ref, aggl_ref, aggu_ref, wl_ref, wu_ref, o_ref):
    o_ref[...] = (af_ref[...]
                  + jnp.dot(aggl_ref[...], wl_ref[...], preferred_element_type=jnp.float32)
                  + jnp.dot(aggu_ref[...], wu_ref[...], preferred_element_type=jnp.float32))


def _outproj(af, aggl, aggu, w_out):
    BA = 2000
    return pl.pallas_call(
        _out_body,
        grid=(N // BA,),
        in_specs=[
            pl.BlockSpec((BA, D), lambda i: (i, 0)),
            pl.BlockSpec((BA, H), lambda i: (i, 0)),
            pl.BlockSpec((BA, H), lambda i: (i, 0)),
            pl.BlockSpec((H, D), lambda i: (0, 0)),
            pl.BlockSpec((H, D), lambda i: (0, 0)),
        ],
        out_specs=pl.BlockSpec((BA, D), lambda i: (i, 0)),
        out_shape=jax.ShapeDtypeStruct((N, D), jnp.float32),
    )(af, aggl, aggu, w_out[:H], w_out[H:])


def kernel(atomic_numbers, edge_index, bond_i, bond_j, embedding, W_angle, W_msg, b_msg, W_out):
    an2d = atomic_numbers.astype(jnp.int32).reshape(N, 1)
    w12 = W_msg[:2 * D]
    w3 = W_msg[2 * D:]

    table, af = _prep(an2d, embedding, w12)

    pad = EP - E
    bi_p = jnp.pad(bond_i, ((0, pad), (0, 0)))
    bj_p = jnp.pad(bond_j, ((0, pad), (0, 0)))
    phh = _angle(bi_p, bj_p, W_angle, w3)

    src_c = jnp.pad(edge_index[0].astype(jnp.int32), (0, pad)).reshape(EP // CH, CH)
    dst_c = jnp.pad(edge_index[1].astype(jnp.int32), (0, pad),
                    constant_values=N).reshape(EP // CH, CH)
    sd_p = jnp.concatenate([src_c, dst_c], axis=1).reshape(-1)
    aggp = _sc_aggregate(sd_p, table, phh, b_msg)

    aggl = aggp[:N]
    aggu = aggp[NP:NP + N]
    return _outproj(af, aggl, aggu, W_out)


# speedup vs baseline: 4.7620x; 1.7756x over previous
import functools
import numpy as np
import jax
import jax.numpy as jnp
from jax import lax
from jax.experimental import pallas as pl
from jax.experimental.pallas import tpu as pltpu
from jax.experimental.pallas import tpu_sc as plsc

N = 10000
E = 160000
D = 256
H = 128
NA = 21
ORDER = 10
MAXEL = 94

NSUB = 16
CH = 64
NCH = 160
NG = NCH // 2
EP = NSUB * CH * NCH
NP = 10112
STRIPE = NP // NSUB

_INV_SQRT_PI = float(1.0 / np.sqrt(np.pi))
_CONST0 = float(1.0 / np.sqrt(2.0) / np.sqrt(np.pi))


def _prep_body(an_ref, embf_ref, embh_ref, w12_ref, t_ref, af_ref):
    an = an_ref[...]
    oh = (lax.broadcasted_iota(jnp.int32, (an.shape[0], MAXEL), 1) == an)
    oh = oh.astype(jnp.float32)
    tw = jnp.dot(embf_ref[...], w12_ref[...], preferred_element_type=jnp.float32)
    t_ref[...] = jnp.dot(oh, tw, preferred_element_type=jnp.float32)
    af_ref[...] = jnp.dot(oh, embh_ref[...], preferred_element_type=jnp.float32)


def _prep(an2d, embedding, w12):
    BA = 2000
    NB = N // BA
    grid = (NB, 2, 2)
    return pl.pallas_call(
        _prep_body,
        grid=grid,
        in_specs=[
            pl.BlockSpec((BA, 1), lambda i, h, t: (i, 0)),
            pl.BlockSpec((MAXEL, D), lambda i, h, t: (0, 0)),
            pl.BlockSpec((MAXEL, H), lambda i, h, t: (0, h)),
            pl.BlockSpec((D, H), lambda i, h, t: (t, h)),
        ],
        out_specs=[
            pl.BlockSpec((BA, H), lambda i, h, t: ((t * 2 + h) * NB + i, 0)),
            pl.BlockSpec((BA, H), lambda i, h, t: (i, h)),
        ],
        out_shape=[
            jax.ShapeDtypeStruct((4 * N, H), jnp.float32),
            jax.ShapeDtypeStruct((N, D), jnp.float32),
        ],
    )(an2d, embedding, embedding, w12)


def _angle_body(bi_ref, bj_ref, wang_ref, w3_ref, p_ref, fea_ref):
    h = pl.program_id(1)

    @pl.when(h == 0)
    def _():
        a = bi_ref[...]
        b = bj_ref[...]
        dp = a[0] * b[0] + a[1] * b[1] + a[2] * b[2]
        ni = jnp.sqrt(a[0] * a[0] + a[1] * a[1] + a[2] * a[2]) + 1e-12
        nj = jnp.sqrt(b[0] * b[0] + b[1] * b[1] + b[2] * b[2]) + 1e-12
        c = dp / (ni * nj) * (1.0 - 1e-6)
        s = jnp.sqrt(jnp.maximum(0.0, 1.0 - c * c))
        fea_ref[0] = jnp.full_like(c, _CONST0)
        ckm1 = jnp.ones_like(c)
        ck = c
        skm1 = jnp.zeros_like(c)
        sk = s
        for n in range(1, ORDER + 1):
            fea_ref[n] = ck * _INV_SQRT_PI
            fea_ref[ORDER + n] = sk * _INV_SQRT_PI
            ckp = 2.0 * c * ck - ckm1
            skp = 2.0 * c * sk - skm1
            ckm1, ck = ck, ckp
            skm1, sk = sk, skp

    wa = jnp.dot(wang_ref[...], w3_ref[...], preferred_element_type=jnp.float32)
    feat = fea_ref[...].reshape(NA, -1)
    p_ref[...] = lax.dot_general(feat, wa, (((0,), (0,)), ((), ())),
                                 preferred_element_type=jnp.float32)


def _angle(bi_t, bj_t, w_angle, w3):
    BE = 2048
    BR = BE // 128
    grid = (EP // BE, 2)
    return pl.pallas_call(
        _angle_body,
        grid=grid,
        in_specs=[
            pl.BlockSpec((3, BR, 128), lambda i, h: (0, i, 0)),
            pl.BlockSpec((3, BR, 128), lambda i, h: (0, i, 0)),
            pl.BlockSpec((NA, D), lambda i, h: (0, 0)),
            pl.BlockSpec((D, H), lambda i, h: (0, h)),
        ],
        out_specs=pl.BlockSpec((BE, H), lambda i, h: (h * (EP // BE) + i, 0)),
        out_shape=jax.ShapeDtypeStruct((2 * EP, H), jnp.float32),
        scratch_shapes=[pltpu.VMEM((NA, BR, 128), jnp.float32)],
    )(bi_t, bj_t, w_angle, w3)


def _sc_body(sd_hbm, t_hbm, p_hbm, b_hbm, out_hbm,
             sdm0, sdm1, dstm0, dstm1, idx0, idx1, rc0, rc1, pb0, pb1, bbuf,
             agg, semA, semB):
    c = lax.axis_index("c")
    s = lax.axis_index("s")

    pltpu.sync_copy(b_hbm.at[pl.ds(c * H, H)], bbuf)

    @plsc.parallel_loop(0, 2 * CH)
    def _z(i):
        for k in range(H // 16):
            rc0[i, pl.ds(k * 16, 16)] = jnp.zeros((16,), jnp.float32)
    for k in range(4):
        pltpu.sync_copy(rc0, agg.at[pl.ds(s * STRIPE + k * 128, 128)])
    pltpu.sync_copy(rc0.at[pl.ds(0, STRIPE - 512)],
                    agg.at[pl.ds(s * STRIPE + 512, STRIPE - 512)])
    plsc.subcore_barrier()

    base1 = c * N
    base2 = 2 * N + c * N

    def prep(j, sdm, dstm, idx, rc, pb, sem):
        g = s * NCH + j
        pltpu.sync_copy(sd_hbm.at[pl.ds(g * 2 * CH, 2 * CH)], sdm.at[0])
        for k in range(CH // 16):
            sl = pl.ds(k * 16, 16)
            idx[sl] = sdm[0, sl] + base1
        for k in range(CH // 16):
            sli = pl.ds(CH + k * 16, 16)
            v = sdm[0, sli]
            idx[sli] = v + base2
            dstm[0, pl.ds(k * 16, 16)] = v
        pltpu.async_copy(t_hbm.at[idx], rc, sem)
        pltpu.async_copy(p_hbm.at[pl.ds(c * EP + g * CH, CH)], pb, sem)

    def drain(rc, pb, sem):
        pltpu.make_async_copy(t_hbm.at[idx0], rc, sem).wait()
        pltpu.make_async_copy(p_hbm.at[pl.ds(0, CH)], pb, sem).wait()

    def compute(rc, pb, dstm):
        @plsc.parallel_loop(0, CH)
        def _cmp(i):
            for k in range(H // 16):
                sl = pl.ds(k * 16, 16)
                x = rc[i, sl] + rc[CH + i, sl] + pb[i, sl] + bbuf[sl]
                rc[i, sl] = x / (1.0 + jnp.exp(-x))
        pltpu.sync_copy(rc.at[pl.ds(0, CH)], agg.at[dstm.at[0]], add=True)

    prep(0, sdm0, dstm0, idx0, rc0, pb0, semA)

    def gbody(g, _):
        j0 = 2 * g
        prep(j0 + 1, sdm1, dstm1, idx1, rc1, pb1, semB)
        drain(rc0, pb0, semA)
        compute(rc0, pb0, dstm0)

        @pl.when(g < NG - 1)
        def _():
            prep(j0 + 2, sdm0, dstm0, idx0, rc0, pb0, semA)
        drain(rc1, pb1, semB)
        compute(rc1, pb1, dstm1)
        return ()

    lax.fori_loop(0, NG, gbody, ())
    plsc.subcore_barrier()

    for k in range(4):
        pltpu.sync_copy(agg.at[pl.ds(s * STRIPE + k * 128, 128)], rc0)
        pltpu.sync_copy(rc0, out_hbm.at[pl.ds(c * NP + s * STRIPE + k * 128, 128)])
    pltpu.sync_copy(agg.at[pl.ds(s * STRIPE + 512, STRIPE - 512)],
                    rc0.at[pl.ds(0, STRIPE - 512)])
    pltpu.sync_copy(rc0.at[pl.ds(0, STRIPE - 512)],
                    out_hbm.at[pl.ds(c * NP + s * STRIPE + 512, STRIPE - 512)])


def _sc_aggregate(sd_p, table, phh, b_msg):
    mesh = plsc.VectorSubcoreMesh(core_axis_name="c", subcore_axis_name="s")
    kern = functools.partial(
        pl.kernel,
        mesh=mesh,
        out_type=jax.ShapeDtypeStruct((2 * NP, H), jnp.float32),
        scratch_types=[
            pltpu.VMEM((1, 2 * CH), jnp.int32),
            pltpu.VMEM((1, 2 * CH), jnp.int32),
            pltpu.VMEM((1, CH), jnp.int32),
            pltpu.VMEM((1, CH), jnp.int32),
            pltpu.VMEM((2 * CH,), jnp.int32),
            pltpu.VMEM((2 * CH,), jnp.int32),
            pltpu.VMEM((2 * CH, H), jnp.float32),
            pltpu.VMEM((2 * CH, H), jnp.float32),
            pltpu.VMEM((CH, H), jnp.float32),
            pltpu.VMEM((CH, H), jnp.float32),
            pltpu.VMEM((H,), jnp.float32),
            pltpu.VMEM_SHARED((NP, H), jnp.float32),
            pltpu.SemaphoreType.DMA,
            pltpu.SemaphoreType.DMA,
        ],
    )(_sc_body)
    return kern(sd_p, table, phh, b_msg)


def _out_body(af_ref, aggl_ref, aggu_ref, wl_ref, wu_ref, o_ref):
    o_ref[...] = (af_ref[...]
                  + jnp.dot(aggl_ref[...], wl_ref[...], preferred_element_type=jnp.float32)
                  + jnp.dot(aggu_ref[...], wu_ref[...], preferred_element_type=jnp.float32))


def _outproj(af, aggl, aggu, w_out):
    BA = 2000
    return pl.pallas_call(
        _out_body,
        grid=(N // BA,),
        in_specs=[
            pl.BlockSpec((BA, D), lambda i: (i, 0)),
            pl.BlockSpec((BA, H), lambda i: (i, 0)),
            pl.BlockSpec((BA, H), lambda i: (i, 0)),
            pl.BlockSpec((H, D), lambda i: (0, 0)),
            pl.BlockSpec((H, D), lambda i: (0, 0)),
        ],
        out_specs=pl.BlockSpec((BA, D), lambda i: (i, 0)),
        out_shape=jax.ShapeDtypeStruct((N, D), jnp.float32),
    )(af, aggl, aggu, w_out[:H], w_out[H:])


def kernel(atomic_numbers, edge_index, bond_i, bond_j, embedding, W_angle, W_msg, b_msg, W_out):
    an2d = atomic_numbers.astype(jnp.int32).reshape(N, 1)
    w12 = W_msg[:2 * D]
    w3 = W_msg[2 * D:]

    table, af = _prep(an2d, embedding, w12)

    pad = EP - E
    bi_t = jnp.pad(bond_i, ((0, pad), (0, 0))).T.reshape(3, EP // 128, 128)
    bj_t = jnp.pad(bond_j, ((0, pad), (0, 0))).T.reshape(3, EP // 128, 128)
    phh = _angle(bi_t, bj_t, W_angle, w3)

    src_c = jnp.pad(edge_index[0].astype(jnp.int32), (0, pad)).reshape(EP // CH, CH)
    dst_c = jnp.pad(edge_index[1].astype(jnp.int32), (0, pad),
                    constant_values=N).reshape(EP // CH, CH)
    sd_p = jnp.concatenate([src_c, dst_c], axis=1).reshape(-1)
    aggp = _sc_aggregate(sd_p, table, phh, b_msg)

    aggl = aggp[:N]
    aggu = aggp[NP:NP + N]
    return _outproj(af, aggl, aggu, W_out)


# speedup vs baseline: 5.0080x; 1.0517x over previous
import functools
import numpy as np
import jax
import jax.numpy as jnp
from jax import lax
from jax.experimental import pallas as pl
from jax.experimental.pallas import tpu as pltpu
from jax.experimental.pallas import tpu_sc as plsc

N = 10000
E = 160000
D = 256
H = 128
NA = 21
ORDER = 10
MAXEL = 94

NSUB = 16
CH = 64
NCH = 160
NG = NCH // 2
EP = NSUB * CH * NCH
NP = 10112
STRIPE = NP // NSUB

_INV_SQRT_PI = float(1.0 / np.sqrt(np.pi))
_CONST0 = float(1.0 / np.sqrt(2.0) / np.sqrt(np.pi))


def _prep_body(an_ref, embf_ref, embh_ref, w12_ref, t_ref, af_ref):
    an = an_ref[...]
    oh = (lax.broadcasted_iota(jnp.int32, (an.shape[0], MAXEL), 1) == an)
    oh = oh.astype(jnp.float32)
    tw = jnp.dot(embf_ref[...], w12_ref[...], preferred_element_type=jnp.float32)
    t_ref[...] = jnp.dot(oh, tw, preferred_element_type=jnp.float32)
    af_ref[...] = jnp.dot(oh, embh_ref[...], preferred_element_type=jnp.float32)


def _prep(an2d, embedding, w12):
    BA = 2000
    NB = N // BA
    grid = (NB, 2, 2)
    return pl.pallas_call(
        _prep_body,
        grid=grid,
        in_specs=[
            pl.BlockSpec((BA, 1), lambda i, h, t: (i, 0)),
            pl.BlockSpec((MAXEL, D), lambda i, h, t: (0, 0)),
            pl.BlockSpec((MAXEL, H), lambda i, h, t: (0, h)),
            pl.BlockSpec((D, H), lambda i, h, t: (t, h)),
        ],
        out_specs=[
            pl.BlockSpec((BA, H), lambda i, h, t: ((t * 2 + h) * NB + i, 0)),
            pl.BlockSpec((BA, H), lambda i, h, t: (i, h)),
        ],
        out_shape=[
            jax.ShapeDtypeStruct((4 * N, H), jnp.float32),
            jax.ShapeDtypeStruct((N, D), jnp.float32),
        ],
    )(an2d, embedding, embedding, w12)


def _angle_body(bi_ref, bj_ref, wang_ref, w3_ref, p_ref, fea_ref):
    h = pl.program_id(1)

    @pl.when(h == 0)
    def _():
        a = bi_ref[...]
        b = bj_ref[...]
        dp = a[0] * b[0] + a[1] * b[1] + a[2] * b[2]
        ni = jnp.sqrt(a[0] * a[0] + a[1] * a[1] + a[2] * a[2]) + 1e-12
        nj = jnp.sqrt(b[0] * b[0] + b[1] * b[1] + b[2] * b[2]) + 1e-12
        c = dp / (ni * nj) * (1.0 - 1e-6)
        s = jnp.sqrt(jnp.maximum(0.0, 1.0 - c * c))
        fea_ref[0] = jnp.full_like(c, _CONST0)
        ckm1 = jnp.ones_like(c)
        ck = c
        skm1 = jnp.zeros_like(c)
        sk = s
        for n in range(1, ORDER + 1):
            fea_ref[n] = ck * _INV_SQRT_PI
            fea_ref[ORDER + n] = sk * _INV_SQRT_PI
            ckp = 2.0 * c * ck - ckm1
            skp = 2.0 * c * sk - skm1
            ckm1, ck = ck, ckp
            skm1, sk = sk, skp

    wa = jnp.dot(wang_ref[...], w3_ref[...], preferred_element_type=jnp.float32)
    feat = fea_ref[...].reshape(NA, -1)
    p_ref[...] = lax.dot_general(feat, wa, (((0,), (0,)), ((), ())),
                                 preferred_element_type=jnp.float32)


def _angle(bi_t, bj_t, w_angle, w3):
    BE = 2048
    BR = BE // 128
    grid = (EP // BE, 2)
    return pl.pallas_call(
        _angle_body,
        grid=grid,
        in_specs=[
            pl.BlockSpec((3, BR, 128), lambda i, h: (0, i, 0)),
            pl.BlockSpec((3, BR, 128), lambda i, h: (0, i, 0)),
            pl.BlockSpec((NA, D), lambda i, h: (0, 0)),
            pl.BlockSpec((D, H), lambda i, h: (0, h)),
        ],
        out_specs=pl.BlockSpec((BE, H), lambda i, h: (h * (EP // BE) + i, 0)),
        out_shape=jax.ShapeDtypeStruct((2 * EP, H), jnp.float32),
        scratch_shapes=[pltpu.VMEM((NA, BR, 128), jnp.float32)],
    )(bi_t, bj_t, w_angle, w3)


def _sc_body(sd_hbm, t_hbm, p_hbm, b_hbm, out_hbm,
             sdm0, sdm1, dstm0, dstm1, idx0, idx1, rc0, rc1, pb0, pb1, bbuf,
             agg, semA, semB, semI):
    c = lax.axis_index("c")
    s = lax.axis_index("s")

    pltpu.sync_copy(b_hbm.at[pl.ds(c * H, H)], bbuf)

    @plsc.parallel_loop(0, 2 * CH)
    def _z(i):
        for k in range(H // 16):
            rc0[i, pl.ds(k * 16, 16)] = jnp.zeros((16,), jnp.float32)
    for k in range(4):
        pltpu.sync_copy(rc0, agg.at[pl.ds(s * STRIPE + k * 128, 128)])
    pltpu.sync_copy(rc0.at[pl.ds(0, STRIPE - 512)],
                    agg.at[pl.ds(s * STRIPE + 512, STRIPE - 512)])
    plsc.subcore_barrier()

    base1 = c * N
    base2 = 2 * N + c * N

    def ids_start(j, sdm):
        g = s * NCH + j
        pltpu.async_copy(sd_hbm.at[pl.ds(g * 2 * CH, 2 * CH)], sdm.at[0], semI)

    def ids_wait(sdm):
        pltpu.make_async_copy(sd_hbm.at[pl.ds(0, 2 * CH)], sdm.at[0], semI).wait()

    def gstart(j, sdm, dstm, idx, rc, pb, sem):
        g = s * NCH + j
        for k in range(CH // 16):
            sl = pl.ds(k * 16, 16)
            idx[sl] = sdm[0, sl] + base1
        for k in range(CH // 16):
            sli = pl.ds(CH + k * 16, 16)
            v = sdm[0, sli]
            idx[sli] = v + base2
            dstm[0, pl.ds(k * 16, 16)] = v
        pltpu.async_copy(t_hbm.at[idx], rc, sem)
        pltpu.async_copy(p_hbm.at[pl.ds(c * EP + g * CH, CH)], pb, sem)

    def drain(rc, pb, sem):
        pltpu.make_async_copy(t_hbm.at[idx0], rc, sem).wait()
        pltpu.make_async_copy(p_hbm.at[pl.ds(0, CH)], pb, sem).wait()

    def compute(rc, pb, dstm):
        @plsc.parallel_loop(0, CH)
        def _cmp(i):
            for k in range(H // 16):
                sl = pl.ds(k * 16, 16)
                x = rc[i, sl] + rc[CH + i, sl] + pb[i, sl] + bbuf[sl]
                rc[i, sl] = x / (1.0 + jnp.exp(-x))
        pltpu.sync_copy(rc.at[pl.ds(0, CH)], agg.at[dstm.at[0]], add=True)

    ids_start(0, sdm0)
    ids_wait(sdm0)
    gstart(0, sdm0, dstm0, idx0, rc0, pb0, semA)
    ids_start(1, sdm1)

    def gbody(g, _):
        j0 = 2 * g
        ids_wait(sdm1)
        gstart(j0 + 1, sdm1, dstm1, idx1, rc1, pb1, semB)

        @pl.when(g < NG - 1)
        def _():
            ids_start(j0 + 2, sdm0)
        drain(rc0, pb0, semA)
        compute(rc0, pb0, dstm0)

        @pl.when(g < NG - 1)
        def _():
            ids_wait(sdm0)
            gstart(j0 + 2, sdm0, dstm0, idx0, rc0, pb0, semA)
            ids_start(j0 + 3, sdm1)
        drain(rc1, pb1, semB)
        compute(rc1, pb1, dstm1)
        return ()

    lax.fori_loop(0, NG, gbody, ())
    plsc.subcore_barrier()

    for k in range(4):
        pltpu.sync_copy(agg.at[pl.ds(s * STRIPE + k * 128, 128)], rc0)
        pltpu.sync_copy(rc0, out_hbm.at[pl.ds(c * NP + s * STRIPE + k * 128, 128)])
    pltpu.sync_copy(agg.at[pl.ds(s * STRIPE + 512, STRIPE - 512)],
                    rc0.at[pl.ds(0, STRIPE - 512)])
    pltpu.sync_copy(rc0.at[pl.ds(0, STRIPE - 512)],
                    out_hbm.at[pl.ds(c * NP + s * STRIPE + 512, STRIPE - 512)])


def _sc_aggregate(sd_p, table, phh, b_msg):
    mesh = plsc.VectorSubcoreMesh(core_axis_name="c", subcore_axis_name="s")
    kern = functools.partial(
        pl.kernel,
        mesh=mesh,
        out_type=jax.ShapeDtypeStruct((2 * NP, H), jnp.float32),
        scratch_types=[
            pltpu.VMEM((1, 2 * CH), jnp.int32),
            pltpu.VMEM((1, 2 * CH), jnp.int32),
            pltpu.VMEM((1, CH), jnp.int32),
            pltpu.VMEM((1, CH), jnp.int32),
            pltpu.VMEM((2 * CH,), jnp.int32),
            pltpu.VMEM((2 * CH,), jnp.int32),
            pltpu.VMEM((2 * CH, H), jnp.float32),
            pltpu.VMEM((2 * CH, H), jnp.float32),
            pltpu.VMEM((CH, H), jnp.float32),
            pltpu.VMEM((CH, H), jnp.float32),
            pltpu.VMEM((H,), jnp.float32),
            pltpu.VMEM_SHARED((NP, H), jnp.float32),
            pltpu.SemaphoreType.DMA,
            pltpu.SemaphoreType.DMA,
            pltpu.SemaphoreType.DMA,
        ],
    )(_sc_body)
    return kern(sd_p, table, phh, b_msg)


def _out_body(af_ref, aggl_ref, aggu_ref, wl_ref, wu_ref, o_ref):
    o_ref[...] = (af_ref[...]
                  + jnp.dot(aggl_ref[...], wl_ref[...], preferred_element_type=jnp.float32)
                  + jnp.dot(aggu_ref[...], wu_ref[...], preferred_element_type=jnp.float32))


def _outproj(af, aggl, aggu, w_out):
    BA = 2000
    return pl.pallas_call(
        _out_body,
        grid=(N // BA,),
        in_specs=[
            pl.BlockSpec((BA, D), lambda i: (i, 0)),
            pl.BlockSpec((BA, H), lambda i: (i, 0)),
            pl.BlockSpec((BA, H), lambda i: (i, 0)),
            pl.BlockSpec((H, D), lambda i: (0, 0)),
            pl.BlockSpec((H, D), lambda i: (0, 0)),
        ],
        out_specs=pl.BlockSpec((BA, D), lambda i: (i, 0)),
        out_shape=jax.ShapeDtypeStruct((N, D), jnp.float32),
    )(af, aggl, aggu, w_out[:H], w_out[H:])


def kernel(atomic_numbers, edge_index, bond_i, bond_j, embedding, W_angle, W_msg, b_msg, W_out):
    an2d = atomic_numbers.astype(jnp.int32).reshape(N, 1)
    w12 = W_msg[:2 * D]
    w3 = W_msg[2 * D:]

    table, af = _prep(an2d, embedding, w12)

    pad = EP - E
    bi_t = jnp.pad(bond_i, ((0, pad), (0, 0))).T.reshape(3, EP // 128, 128)
    bj_t = jnp.pad(bond_j, ((0, pad), (0, 0))).T.reshape(3, EP // 128, 128)
    phh = _angle(bi_t, bj_t, W_angle, w3)

    src_c = jnp.pad(edge_index[0].astype(jnp.int32), (0, pad)).reshape(EP // CH, CH)
    dst_c = jnp.pad(edge_index[1].astype(jnp.int32), (0, pad),
                    constant_values=N).reshape(EP // CH, CH)
    sd_p = jnp.concatenate([src_c, dst_c], axis=1).reshape(-1)
    aggp = _sc_aggregate(sd_p, table, phh, b_msg)

    aggl = aggp[:N]
    aggu = aggp[NP:NP + N]
    return _outproj(af, aggl, aggu, W_out)


# speedup vs baseline: 5.1786x; 1.0341x over previous
import functools
import numpy as np
import jax
import jax.numpy as jnp
from jax import lax
from jax.experimental import pallas as pl
from jax.experimental.pallas import tpu as pltpu
from jax.experimental.pallas import tpu_sc as plsc

N = 10000
E = 160000
D = 256
H = 128
NA = 21
ORDER = 10
MAXEL = 94

NSUB = 16
CH = 64
NCH = 160
NG = NCH // 2
EP = NSUB * CH * NCH
NP = 10112
STRIPE = NP // NSUB

_INV_SQRT_PI = float(1.0 / np.sqrt(np.pi))
_CONST0 = float(1.0 / np.sqrt(2.0) / np.sqrt(np.pi))


def _prep_body(an_ref, embf_ref, embh_ref, w12_ref, t_ref, af_ref):
    an = an_ref[...]
    oh = (lax.broadcasted_iota(jnp.int32, (an.shape[0], MAXEL), 1) == an)
    oh = oh.astype(jnp.float32)
    tw = jnp.dot(embf_ref[...], w12_ref[...], preferred_element_type=jnp.float32)
    t_ref[...] = jnp.dot(oh, tw, preferred_element_type=jnp.float32)
    af_ref[...] = jnp.dot(oh, embh_ref[...], preferred_element_type=jnp.float32)


def _prep(an2d, embedding, w12):
    BA = 2000
    NB = N // BA
    grid = (NB, 2, 2)
    return pl.pallas_call(
        _prep_body,
        grid=grid,
        in_specs=[
            pl.BlockSpec((BA, 1), lambda i, h, t: (i, 0)),
            pl.BlockSpec((MAXEL, D), lambda i, h, t: (0, 0)),
            pl.BlockSpec((MAXEL, H), lambda i, h, t: (0, h)),
            pl.BlockSpec((D, H), lambda i, h, t: (t, h)),
        ],
        out_specs=[
            pl.BlockSpec((BA, H), lambda i, h, t: ((t * 2 + h) * NB + i, 0)),
            pl.BlockSpec((BA, H), lambda i, h, t: (i, h)),
        ],
        out_shape=[
            jax.ShapeDtypeStruct((4 * N, H), jnp.float32),
            jax.ShapeDtypeStruct((N, D), jnp.float32),
        ],
    )(an2d, embedding, embedding, w12)


def _angle_body(bi_ref, bj_ref, wang_ref, w3_ref, b_ref, p_ref, fea_ref):
    h = pl.program_id(1)

    @pl.when(h == 0)
    def _():
        a = bi_ref[...]
        b = bj_ref[...]
        dp = a[0] * b[0] + a[1] * b[1] + a[2] * b[2]
        ni = jnp.sqrt(a[0] * a[0] + a[1] * a[1] + a[2] * a[2]) + 1e-12
        nj = jnp.sqrt(b[0] * b[0] + b[1] * b[1] + b[2] * b[2]) + 1e-12
        c = dp / (ni * nj) * (1.0 - 1e-6)
        s = jnp.sqrt(jnp.maximum(0.0, 1.0 - c * c))
        fea_ref[0] = jnp.full_like(c, _CONST0)
        ckm1 = jnp.ones_like(c)
        ck = c
        skm1 = jnp.zeros_like(c)
        sk = s
        for n in range(1, ORDER + 1):
            fea_ref[n] = ck * _INV_SQRT_PI
            fea_ref[ORDER + n] = sk * _INV_SQRT_PI
            ckp = 2.0 * c * ck - ckm1
            skp = 2.0 * c * sk - skm1
            ckm1, ck = ck, ckp
            skm1, sk = sk, skp

    wa = jnp.dot(wang_ref[...], w3_ref[...], preferred_element_type=jnp.float32)
    feat = fea_ref[...].reshape(NA, -1)
    p_ref[...] = lax.dot_general(feat, wa, (((0,), (0,)), ((), ())),
                                 preferred_element_type=jnp.float32) + b_ref[...][0]


def _angle(bi_t, bj_t, w_angle, w3, b2):
    BE = 2048
    BR = BE // 128
    grid = (EP // BE, 2)
    return pl.pallas_call(
        _angle_body,
        grid=grid,
        in_specs=[
            pl.BlockSpec((3, BR, 128), lambda i, h: (0, i, 0)),
            pl.BlockSpec((3, BR, 128), lambda i, h: (0, i, 0)),
            pl.BlockSpec((NA, D), lambda i, h: (0, 0)),
            pl.BlockSpec((D, H), lambda i, h: (0, h)),
            pl.BlockSpec((1, 1, H), lambda i, h: (h, 0, 0)),
        ],
        out_specs=pl.BlockSpec((BE, H), lambda i, h: (h * (EP // BE) + i, 0)),
        out_shape=jax.ShapeDtypeStruct((2 * EP, H), jnp.float32),
        scratch_shapes=[pltpu.VMEM((NA, BR, 128), jnp.float32)],
    )(bi_t, bj_t, w_angle, w3, b2)


def _sc_body(sd_hbm, t_hbm, p_hbm, out_hbm,
             sdm0, sdm1, dstm0, dstm1, idx0, idx1, rc0, rc1, pb0, pb1,
             agg, semA, semB, semI):
    c = lax.axis_index("c")
    s = lax.axis_index("s")

    @plsc.parallel_loop(0, 2 * CH)
    def _z(i):
        for k in range(H // 16):
            rc0[i, pl.ds(k * 16, 16)] = jnp.zeros((16,), jnp.float32)
    for k in range(4):
        pltpu.sync_copy(rc0, agg.at[pl.ds(s * STRIPE + k * 128, 128)])
    pltpu.sync_copy(rc0.at[pl.ds(0, STRIPE - 512)],
                    agg.at[pl.ds(s * STRIPE + 512, STRIPE - 512)])
    plsc.subcore_barrier()

    base1 = c * N
    base2 = 2 * N + c * N

    def ids_start(j, sdm):
        g = s * NCH + j
        pltpu.async_copy(sd_hbm.at[pl.ds(g * 2 * CH, 2 * CH)], sdm.at[0], semI)

    def ids_wait(sdm):
        pltpu.make_async_copy(sd_hbm.at[pl.ds(0, 2 * CH)], sdm.at[0], semI).wait()

    def gstart(j, sdm, dstm, idx, rc, pb, sem):
        g = s * NCH + j
        for k in range(CH // 16):
            sl = pl.ds(k * 16, 16)
            idx[sl] = sdm[0, sl] + base1
        for k in range(CH // 16):
            sli = pl.ds(CH + k * 16, 16)
            v = sdm[0, sli]
            idx[sli] = v + base2
            dstm[0, pl.ds(k * 16, 16)] = v
        pltpu.async_copy(t_hbm.at[idx], rc, sem)
        pltpu.async_copy(p_hbm.at[pl.ds(c * EP + g * CH, CH)], pb, sem)

    def drain(rc, pb, sem):
        pltpu.make_async_copy(t_hbm.at[idx0], rc, sem).wait()
        pltpu.make_async_copy(p_hbm.at[pl.ds(0, CH)], pb, sem).wait()

    def compute(rc, pb, dstm):
        @plsc.parallel_loop(0, CH)
        def _cmp(i):
            for k in range(H // 16):
                sl = pl.ds(k * 16, 16)
                x = rc[i, sl] + rc[CH + i, sl] + pb[i, sl]
                rc[i, sl] = x - x / (1.0 + jnp.exp(x))
        pltpu.sync_copy(rc.at[pl.ds(0, CH)], agg.at[dstm.at[0]], add=True)

    ids_start(0, sdm0)
    ids_wait(sdm0)
    gstart(0, sdm0, dstm0, idx0, rc0, pb0, semA)
    ids_start(1, sdm1)

    def gbody(g, _):
        j0 = 2 * g
        ids_wait(sdm1)
        gstart(j0 + 1, sdm1, dstm1, idx1, rc1, pb1, semB)

        @pl.when(g < NG - 1)
        def _():
            ids_start(j0 + 2, sdm0)
        drain(rc0, pb0, semA)
        compute(rc0, pb0, dstm0)

        @pl.when(g < NG - 1)
        def _():
            ids_wait(sdm0)
            gstart(j0 + 2, sdm0, dstm0, idx0, rc0, pb0, semA)
            ids_start(j0 + 3, sdm1)
        drain(rc1, pb1, semB)
        compute(rc1, pb1, dstm1)
        return ()

    lax.fori_loop(0, NG, gbody, ())
    plsc.subcore_barrier()

    for k in range(4):
        pltpu.sync_copy(agg.at[pl.ds(s * STRIPE + k * 128, 128)], rc0)
        pltpu.sync_copy(rc0, out_hbm.at[pl.ds(c * NP + s * STRIPE + k * 128, 128)])
    pltpu.sync_copy(agg.at[pl.ds(s * STRIPE + 512, STRIPE - 512)],
                    rc0.at[pl.ds(0, STRIPE - 512)])
    pltpu.sync_copy(rc0.at[pl.ds(0, STRIPE - 512)],
                    out_hbm.at[pl.ds(c * NP + s * STRIPE + 512, STRIPE - 512)])


def _sc_aggregate(sd_p, table, phh):
    mesh = plsc.VectorSubcoreMesh(core_axis_name="c", subcore_axis_name="s")
    kern = functools.partial(
        pl.kernel,
        mesh=mesh,
        out_type=jax.ShapeDtypeStruct((2 * NP, H), jnp.float32),
        scratch_types=[
            pltpu.VMEM((1, 2 * CH), jnp.int32),
            pltpu.VMEM((1, 2 * CH), jnp.int32),
            pltpu.VMEM((1, CH), jnp.int32),
            pltpu.VMEM((1, CH), jnp.int32),
            pltpu.VMEM((2 * CH,), jnp.int32),
            pltpu.VMEM((2 * CH,), jnp.int32),
            pltpu.VMEM((2 * CH, H), jnp.float32),
            pltpu.VMEM((2 * CH, H), jnp.float32),
            pltpu.VMEM((CH, H), jnp.float32),
            pltpu.VMEM((CH, H), jnp.float32),
            pltpu.VMEM_SHARED((NP, H), jnp.float32),
            pltpu.SemaphoreType.DMA,
            pltpu.SemaphoreType.DMA,
            pltpu.SemaphoreType.DMA,
        ],
    )(_sc_body)
    return kern(sd_p, table, phh)


def _out_body(af_ref, aggl_ref, aggu_ref, wl_ref, wu_ref, o_ref):
    o_ref[...] = (af_ref[...]
                  + jnp.dot(aggl_ref[...], wl_ref[...], preferred_element_type=jnp.float32)
                  + jnp.dot(aggu_ref[...], wu_ref[...], preferred_element_type=jnp.float32))


def _outproj(af, aggl, aggu, w_out):
    BA = 2000
    return pl.pallas_call(
        _out_body,
        grid=(N // BA,),
        in_specs=[
            pl.BlockSpec((BA, D), lambda i: (i, 0)),
            pl.BlockSpec((BA, H), lambda i: (i, 0)),
            pl.BlockSpec((BA, H), lambda i: (i, 0)),
            pl.BlockSpec((H, D), lambda i: (0, 0)),
            pl.BlockSpec((H, D), lambda i: (0, 0)),
        ],
        out_specs=pl.BlockSpec((BA, D), lambda i: (i, 0)),
        out_shape=jax.ShapeDtypeStruct((N, D), jnp.float32),
    )(af, aggl, aggu, w_out[:H], w_out[H:])


def kernel(atomic_numbers, edge_index, bond_i, bond_j, embedding, W_angle, W_msg, b_msg, W_out):
    an2d = atomic_numbers.astype(jnp.int32).reshape(N, 1)
    w12 = W_msg[:2 * D]
    w3 = W_msg[2 * D:]

    table, af = _prep(an2d, embedding, w12)

    pad = EP - E
    bi_t = jnp.pad(bond_i, ((0, pad), (0, 0))).T.reshape(3, EP // 128, 128)
    bj_t = jnp.pad(bond_j, ((0, pad), (0, 0))).T.reshape(3, EP // 128, 128)
    phh = _angle(bi_t, bj_t, W_angle, w3, b_msg.reshape(2, 1, H))

    src_c = jnp.pad(edge_index[0].astype(jnp.int32), (0, pad)).reshape(EP // CH, CH)
    dst_c = jnp.pad(edge_index[1].astype(jnp.int32), (0, pad),
                    constant_values=N).reshape(EP // CH, CH)
    sd_p = jnp.concatenate([src_c, dst_c], axis=1).reshape(-1)
    aggp = _sc_aggregate(sd_p, table, phh)

    aggl = aggp[:N]
    aggu = aggp[NP:NP + N]
    return _outproj(af, aggl, aggu, W_out)


# speedup vs baseline: 5.6102x; 1.0834x over previous
import functools
import numpy as np
import jax
import jax.numpy as jnp
from jax import lax
from jax.experimental import pallas as pl
from jax.experimental.pallas import tpu as pltpu
from jax.experimental.pallas import tpu_sc as plsc

N = 10000
E = 160000
D = 256
H = 128
NA = 21
ORDER = 10
MAXEL = 94

NSUB = 16
CH = 64
NCH = 160
NG = NCH // 2
EP = NSUB * CH * NCH
NP = 10112
STRIPE = NP // NSUB

_INV_SQRT_PI = float(1.0 / np.sqrt(np.pi))
_CONST0 = float(1.0 / np.sqrt(2.0) / np.sqrt(np.pi))


def _prep_body(an_ref, embf_ref, embh_ref, w12_ref, t_ref, af_ref):
    an = an_ref[...]
    oh = (lax.broadcasted_iota(jnp.int32, (an.shape[0], MAXEL), 1) == an)
    oh = oh.astype(jnp.float32)
    tw = jnp.dot(embf_ref[...], w12_ref[...], preferred_element_type=jnp.float32)
    t_ref[...] = jnp.dot(oh, tw, preferred_element_type=jnp.float32)
    af_ref[...] = jnp.dot(oh, embh_ref[...], preferred_element_type=jnp.float32)


def _prep(an2d, embedding, w12):
    BA = 2000
    NB = N // BA
    grid = (NB, 2, 2)
    return pl.pallas_call(
        _prep_body,
        grid=grid,
        in_specs=[
            pl.BlockSpec((BA, 1), lambda i, h, t: (i, 0)),
            pl.BlockSpec((MAXEL, D), lambda i, h, t: (0, 0)),
            pl.BlockSpec((MAXEL, H), lambda i, h, t: (0, h)),
            pl.BlockSpec((D, H), lambda i, h, t: (t, h)),
        ],
        out_specs=[
            pl.BlockSpec((BA, H), lambda i, h, t: ((t * 2 + h) * NB + i, 0)),
            pl.BlockSpec((BA, H), lambda i, h, t: (i, h)),
        ],
        out_shape=[
            jax.ShapeDtypeStruct((4 * N, H), jnp.float32),
            jax.ShapeDtypeStruct((N, D), jnp.float32),
        ],
    )(an2d, embedding, embedding, w12)


def _angle_body(bi_ref, bj_ref, wang_ref, w3_ref, b_ref, p_ref, fea_ref):
    h = pl.program_id(1)

    @pl.when(h == 0)
    def _():
        a = bi_ref[...]
        b = bj_ref[...]
        dp = a[0] * b[0] + a[1] * b[1] + a[2] * b[2]
        ni = jnp.sqrt(a[0] * a[0] + a[1] * a[1] + a[2] * a[2]) + 1e-12
        nj = jnp.sqrt(b[0] * b[0] + b[1] * b[1] + b[2] * b[2]) + 1e-12
        c = dp / (ni * nj) * (1.0 - 1e-6)
        s = jnp.sqrt(jnp.maximum(0.0, 1.0 - c * c))
        fea_ref[0] = jnp.full_like(c, _CONST0)
        ckm1 = jnp.ones_like(c)
        ck = c
        skm1 = jnp.zeros_like(c)
        sk = s
        for n in range(1, ORDER + 1):
            fea_ref[n] = ck * _INV_SQRT_PI
            fea_ref[ORDER + n] = sk * _INV_SQRT_PI
            ckp = 2.0 * c * ck - ckm1
            skp = 2.0 * c * sk - skm1
            ckm1, ck = ck, ckp
            skm1, sk = sk, skp

    wa = jnp.dot(wang_ref[...], w3_ref[...], preferred_element_type=jnp.float32)
    feat = fea_ref[...].reshape(NA, -1)
    p_ref[...] = lax.dot_general(feat, wa, (((0,), (0,)), ((), ())),
                                 preferred_element_type=jnp.float32) + b_ref[...][0]


def _angle(bi_t, bj_t, w_angle, w3, b2):
    BE = 4096
    BR = BE // 128
    grid = (EP // BE, 2)
    return pl.pallas_call(
        _angle_body,
        grid=grid,
        in_specs=[
            pl.BlockSpec((3, BR, 128), lambda i, h: (0, i, 0)),
            pl.BlockSpec((3, BR, 128), lambda i, h: (0, i, 0)),
            pl.BlockSpec((NA, D), lambda i, h: (0, 0)),
            pl.BlockSpec((D, H), lambda i, h: (0, h)),
            pl.BlockSpec((1, 1, H), lambda i, h: (h, 0, 0)),
        ],
        out_specs=pl.BlockSpec((BE, H), lambda i, h: (h * (EP // BE) + i, 0)),
        out_shape=jax.ShapeDtypeStruct((2 * EP, H), jnp.float32),
        scratch_shapes=[pltpu.VMEM((NA, BR, 128), jnp.float32)],
    )(bi_t, bj_t, w_angle, w3, b2)


def _sc_body(sd_hbm, t_hbm, p_hbm, out_hbm,
             sdm0, sdm1, dstm0, dstm1, idx0, idx1, rc0, rc1, pb0, pb1,
             agg, semA, semB, semI):
    c = lax.axis_index("c")
    s = lax.axis_index("s")

    @plsc.parallel_loop(0, 2 * CH)
    def _z(i):
        for k in range(H // 16):
            rc0[i, pl.ds(k * 16, 16)] = jnp.zeros((16,), jnp.float32)
    for k in range(4):
        pltpu.sync_copy(rc0, agg.at[pl.ds(s * STRIPE + k * 128, 128)])
    pltpu.sync_copy(rc0.at[pl.ds(0, STRIPE - 512)],
                    agg.at[pl.ds(s * STRIPE + 512, STRIPE - 512)])
    plsc.subcore_barrier()

    base1 = c * N
    base2 = 2 * N + c * N

    def ids_start(j, sdm):
        g = s * NCH + j
        pltpu.async_copy(sd_hbm.at[pl.ds(g * 2 * CH, 2 * CH)], sdm.at[0], semI)

    def ids_wait(sdm):
        pltpu.make_async_copy(sd_hbm.at[pl.ds(0, 2 * CH)], sdm.at[0], semI).wait()

    def gstart(j, sdm, dstm, idx, rc, pb, sem):
        g = s * NCH + j
        for k in range(CH // 16):
            sl = pl.ds(k * 16, 16)
            idx[sl] = sdm[0, sl] + base1
        for k in range(CH // 16):
            sli = pl.ds(CH + k * 16, 16)
            v = sdm[0, sli]
            idx[sli] = v + base2
            dstm[0, pl.ds(k * 16, 16)] = v
        pltpu.async_copy(t_hbm.at[idx], rc, sem)
        pltpu.async_copy(p_hbm.at[pl.ds(c * EP + g * CH, CH)], pb, sem)

    def drain(rc, pb, sem):
        pltpu.make_async_copy(t_hbm.at[idx0], rc, sem).wait()
        pltpu.make_async_copy(p_hbm.at[pl.ds(0, CH)], pb, sem).wait()

    def compute(rc, pb, dstm):
        @plsc.parallel_loop(0, CH)
        def _cmp(i):
            for k in range(H // 16):
                sl = pl.ds(k * 16, 16)
                x = rc[i, sl] + rc[CH + i, sl] + pb[i, sl]
                rc[i, sl] = x - x / (1.0 + jnp.exp(x))
        pltpu.sync_copy(rc.at[pl.ds(0, CH)], agg.at[dstm.at[0]], add=True)

    ids_start(0, sdm0)
    ids_wait(sdm0)
    gstart(0, sdm0, dstm0, idx0, rc0, pb0, semA)
    ids_start(1, sdm1)

    def gbody(g, _):
        j0 = 2 * g
        ids_wait(sdm1)
        gstart(j0 + 1, sdm1, dstm1, idx1, rc1, pb1, semB)

        @pl.when(g < NG - 1)
        def _():
            ids_start(j0 + 2, sdm0)
        drain(rc0, pb0, semA)
        compute(rc0, pb0, dstm0)

        @pl.when(g < NG - 1)
        def _():
            ids_wait(sdm0)
            gstart(j0 + 2, sdm0, dstm0, idx0, rc0, pb0, semA)
            ids_start(j0 + 3, sdm1)
        drain(rc1, pb1, semB)
        compute(rc1, pb1, dstm1)
        return ()

    lax.fori_loop(0, NG, gbody, ())
    plsc.subcore_barrier()

    for k in range(4):
        pltpu.sync_copy(agg.at[pl.ds(s * STRIPE + k * 128, 128)], rc0)
        pltpu.sync_copy(rc0, out_hbm.at[pl.ds(c * NP + s * STRIPE + k * 128, 128)])
    pltpu.sync_copy(agg.at[pl.ds(s * STRIPE + 512, STRIPE - 512)],
                    rc0.at[pl.ds(0, STRIPE - 512)])
    pltpu.sync_copy(rc0.at[pl.ds(0, STRIPE - 512)],
                    out_hbm.at[pl.ds(c * NP + s * STRIPE + 512, STRIPE - 512)])


def _sc_aggregate(sd_p, table, phh):
    mesh = plsc.VectorSubcoreMesh(core_axis_name="c", subcore_axis_name="s")
    kern = functools.partial(
        pl.kernel,
        mesh=mesh,
        out_type=jax.ShapeDtypeStruct((2 * NP, H), jnp.float32),
        scratch_types=[
            pltpu.VMEM((1, 2 * CH), jnp.int32),
            pltpu.VMEM((1, 2 * CH), jnp.int32),
            pltpu.VMEM((1, CH), jnp.int32),
            pltpu.VMEM((1, CH), jnp.int32),
            pltpu.VMEM((2 * CH,), jnp.int32),
            pltpu.VMEM((2 * CH,), jnp.int32),
            pltpu.VMEM((2 * CH, H), jnp.float32),
            pltpu.VMEM((2 * CH, H), jnp.float32),
            pltpu.VMEM((CH, H), jnp.float32),
            pltpu.VMEM((CH, H), jnp.float32),
            pltpu.VMEM_SHARED((NP, H), jnp.float32),
            pltpu.SemaphoreType.DMA,
            pltpu.SemaphoreType.DMA,
            pltpu.SemaphoreType.DMA,
        ],
    )(_sc_body)
    return kern(sd_p, table, phh)


def _out_body(af_ref, aggl_ref, aggu_ref, wl_ref, wu_ref, o_ref):
    o_ref[...] = (af_ref[...]
                  + jnp.dot(aggl_ref[...], wl_ref[...], preferred_element_type=jnp.float32)
                  + jnp.dot(aggu_ref[...], wu_ref[...], preferred_element_type=jnp.float32))


def _outproj(af, aggl, aggu, w_out):
    BA = 2000
    return pl.pallas_call(
        _out_body,
        grid=(N // BA,),
        in_specs=[
            pl.BlockSpec((BA, D), lambda i: (i, 0)),
            pl.BlockSpec((BA, H), lambda i: (i, 0)),
            pl.BlockSpec((BA, H), lambda i: (i, 0)),
            pl.BlockSpec((H, D), lambda i: (0, 0)),
            pl.BlockSpec((H, D), lambda i: (0, 0)),
        ],
        out_specs=pl.BlockSpec((BA, D), lambda i: (i, 0)),
        out_shape=jax.ShapeDtypeStruct((N, D), jnp.float32),
    )(af, aggl, aggu, w_out[:H], w_out[H:])


def kernel(atomic_numbers, edge_index, bond_i, bond_j, embedding, W_angle, W_msg, b_msg, W_out):
    an2d = atomic_numbers.astype(jnp.int32).reshape(N, 1)
    w12 = W_msg[:2 * D]
    w3 = W_msg[2 * D:]

    table, af = _prep(an2d, embedding, w12)

    pad = EP - E
    bi_t = jnp.pad(bond_i, ((0, pad), (0, 0))).T.reshape(3, EP // 128, 128)
    bj_t = jnp.pad(bond_j, ((0, pad), (0, 0))).T.reshape(3, EP // 128, 128)
    phh = _angle(bi_t, bj_t, W_angle, w3, b_msg.reshape(2, 1, H))

    src_c = jnp.pad(edge_index[0].astype(jnp.int32), (0, pad)).reshape(EP // CH, CH)
    dst_c = jnp.pad(edge_index[1].astype(jnp.int32), (0, pad),
                    constant_values=N).reshape(EP // CH, CH)
    sd_p = jnp.concatenate([src_c, dst_c], axis=1).reshape(-1)
    aggp = _sc_aggregate(sd_p, table, phh)

    aggl = aggp[:N]
    aggu = aggp[NP:NP + N]
    return _outproj(af, aggl, aggu, W_out)


# speedup vs baseline: 5.8423x; 1.0414x over previous
import functools
import numpy as np
import jax
import jax.numpy as jnp
from jax import lax
from jax.experimental import pallas as pl
from jax.experimental.pallas import tpu as pltpu
from jax.experimental.pallas import tpu_sc as plsc

N = 10000
E = 160000
D = 256
H = 128
NA = 21
ORDER = 10
MAXEL = 94

NSUB = 16
CH = 64
NCH = 160
NG = NCH // 2
EP = NSUB * CH * NCH
NP = 10112
STRIPE = NP // NSUB

_INV_SQRT_PI = float(1.0 / np.sqrt(np.pi))
_CONST0 = float(1.0 / np.sqrt(2.0) / np.sqrt(np.pi))


def _prep_body(an_ref, embf_ref, embh_ref, w12_ref, t_ref, af_ref):
    an = an_ref[...]
    oh = (lax.broadcasted_iota(jnp.int32, (an.shape[0], MAXEL), 1) == an)
    oh = oh.astype(jnp.float32)
    tw = jnp.dot(embf_ref[...], w12_ref[...], preferred_element_type=jnp.float32)
    t_ref[...] = jnp.dot(oh, tw, preferred_element_type=jnp.float32)
    af_ref[...] = jnp.dot(oh, embh_ref[...], preferred_element_type=jnp.float32)


def _prep(an2d, embedding, w12):
    BA = 2000
    NB = N // BA
    grid = (NB, 2, 2)
    return pl.pallas_call(
        _prep_body,
        grid=grid,
        in_specs=[
            pl.BlockSpec((BA, 1), lambda i, h, t: (i, 0)),
            pl.BlockSpec((MAXEL, D), lambda i, h, t: (0, 0)),
            pl.BlockSpec((MAXEL, H), lambda i, h, t: (0, h)),
            pl.BlockSpec((D, H), lambda i, h, t: (t, h)),
        ],
        out_specs=[
            pl.BlockSpec((BA, H), lambda i, h, t: ((t * 2 + h) * NB + i, 0)),
            pl.BlockSpec((BA, H), lambda i, h, t: (i, h)),
        ],
        out_shape=[
            jax.ShapeDtypeStruct((4 * N, H), jnp.float32),
            jax.ShapeDtypeStruct((N, D), jnp.float32),
        ],
    )(an2d, embedding, embedding, w12)


def _angle_body(bi_ref, bj_ref, wang_ref, w3_ref, b_ref, p_ref, fea_ref):
    h = pl.program_id(1)

    @pl.when(h == 0)
    def _():
        a = bi_ref[...]
        b = bj_ref[...]
        dp = a[0] * b[0] + a[1] * b[1] + a[2] * b[2]
        ni = jnp.sqrt(a[0] * a[0] + a[1] * a[1] + a[2] * a[2]) + 1e-12
        nj = jnp.sqrt(b[0] * b[0] + b[1] * b[1] + b[2] * b[2]) + 1e-12
        c = dp / (ni * nj) * (1.0 - 1e-6)
        s = jnp.sqrt(jnp.maximum(0.0, 1.0 - c * c))
        fea_ref[0] = jnp.full_like(c, _CONST0)
        ckm1 = jnp.ones_like(c)
        ck = c
        skm1 = jnp.zeros_like(c)
        sk = s
        for n in range(1, ORDER + 1):
            fea_ref[n] = ck * _INV_SQRT_PI
            fea_ref[ORDER + n] = sk * _INV_SQRT_PI
            ckp = 2.0 * c * ck - ckm1
            skp = 2.0 * c * sk - skm1
            ckm1, ck = ck, ckp
            skm1, sk = sk, skp

    wa = jnp.dot(wang_ref[...], w3_ref[...], preferred_element_type=jnp.float32)
    feat = fea_ref[...].reshape(NA, -1)
    p_ref[...] = lax.dot_general(feat, wa, (((0,), (0,)), ((), ())),
                                 preferred_element_type=jnp.float32) + b_ref[...][0]


def _angle(bi_t, bj_t, w_angle, w3, b2):
    BE = 8192
    BR = BE // 128
    grid = (EP // BE, 2)
    return pl.pallas_call(
        _angle_body,
        grid=grid,
        in_specs=[
            pl.BlockSpec((3, BR, 128), lambda i, h: (0, i, 0)),
            pl.BlockSpec((3, BR, 128), lambda i, h: (0, i, 0)),
            pl.BlockSpec((NA, D), lambda i, h: (0, 0)),
            pl.BlockSpec((D, H), lambda i, h: (0, h)),
            pl.BlockSpec((1, 1, H), lambda i, h: (h, 0, 0)),
        ],
        out_specs=pl.BlockSpec((BE, H), lambda i, h: (h * (EP // BE) + i, 0)),
        out_shape=jax.ShapeDtypeStruct((2 * EP, H), jnp.float32),
        scratch_shapes=[pltpu.VMEM((NA, BR, 128), jnp.float32)],
    )(bi_t, bj_t, w_angle, w3, b2)


def _sc_body(sd_hbm, t_hbm, p_hbm, out_hbm,
             sdm0, sdm1, dstm0, dstm1, idx0, idx1, rc0, rc1, pb0, pb1,
             agg, semA, semB, semI):
    c = lax.axis_index("c")
    s = lax.axis_index("s")

    @plsc.parallel_loop(0, 2 * CH)
    def _z(i):
        for k in range(H // 16):
            rc0[i, pl.ds(k * 16, 16)] = jnp.zeros((16,), jnp.float32)
    for k in range(4):
        pltpu.sync_copy(rc0, agg.at[pl.ds(s * STRIPE + k * 128, 128)])
    pltpu.sync_copy(rc0.at[pl.ds(0, STRIPE - 512)],
                    agg.at[pl.ds(s * STRIPE + 512, STRIPE - 512)])
    plsc.subcore_barrier()

    base1 = c * N
    base2 = 2 * N + c * N

    def ids_start(j, sdm):
        g = s * NCH + j
        pltpu.async_copy(sd_hbm.at[pl.ds(g * 2 * CH, 2 * CH)], sdm.at[0], semI)

    def ids_wait(sdm):
        pltpu.make_async_copy(sd_hbm.at[pl.ds(0, 2 * CH)], sdm.at[0], semI).wait()

    def gstart(j, sdm, dstm, idx, rc, pb, sem):
        g = s * NCH + j
        for k in range(CH // 16):
            sl = pl.ds(k * 16, 16)
            idx[sl] = sdm[0, sl] + base1
        for k in range(CH // 16):
            sli = pl.ds(CH + k * 16, 16)
            v = sdm[0, sli]
            idx[sli] = v + base2
            dstm[0, pl.ds(k * 16, 16)] = v
        pltpu.async_copy(t_hbm.at[idx], rc, sem)
        pltpu.async_copy(p_hbm.at[pl.ds(c * EP + g * CH, CH)], pb, sem)

    def drain(rc, pb, sem):
        pltpu.make_async_copy(t_hbm.at[idx0], rc, sem).wait()
        pltpu.make_async_copy(p_hbm.at[pl.ds(0, CH)], pb, sem).wait()

    def compute(rc, pb, dstm):
        @plsc.parallel_loop(0, CH)
        def _cmp(i):
            for k in range(H // 16):
                sl = pl.ds(k * 16, 16)
                x = rc[i, sl] + rc[CH + i, sl] + pb[i, sl]
                rc[i, sl] = x - x / (1.0 + jnp.exp(x))
        pltpu.sync_copy(rc.at[pl.ds(0, CH)], agg.at[dstm.at[0]], add=True)

    ids_start(0, sdm0)
    ids_wait(sdm0)
    gstart(0, sdm0, dstm0, idx0, rc0, pb0, semA)
    ids_start(1, sdm1)

    def gbody(g, _):
        j0 = 2 * g
        ids_wait(sdm1)
        gstart(j0 + 1, sdm1, dstm1, idx1, rc1, pb1, semB)

        @pl.when(g < NG - 1)
        def _():
            ids_start(j0 + 2, sdm0)
        drain(rc0, pb0, semA)
        compute(rc0, pb0, dstm0)

        @pl.when(g < NG - 1)
        def _():
            ids_wait(sdm0)
            gstart(j0 + 2, sdm0, dstm0, idx0, rc0, pb0, semA)
            ids_start(j0 + 3, sdm1)
        drain(rc1, pb1, semB)
        compute(rc1, pb1, dstm1)
        return ()

    lax.fori_loop(0, NG, gbody, ())
    plsc.subcore_barrier()

    for k in range(4):
        pltpu.sync_copy(agg.at[pl.ds(s * STRIPE + k * 128, 128)], rc0)
        pltpu.sync_copy(rc0, out_hbm.at[pl.ds(c * NP + s * STRIPE + k * 128, 128)])
    pltpu.sync_copy(agg.at[pl.ds(s * STRIPE + 512, STRIPE - 512)],
                    rc0.at[pl.ds(0, STRIPE - 512)])
    pltpu.sync_copy(rc0.at[pl.ds(0, STRIPE - 512)],
                    out_hbm.at[pl.ds(c * NP + s * STRIPE + 512, STRIPE - 512)])


def _sc_aggregate(sd_p, table, phh):
    mesh = plsc.VectorSubcoreMesh(core_axis_name="c", subcore_axis_name="s")
    kern = functools.partial(
        pl.kernel,
        mesh=mesh,
        out_type=jax.ShapeDtypeStruct((2 * NP, H), jnp.float32),
        scratch_types=[
            pltpu.VMEM((1, 2 * CH), jnp.int32),
            pltpu.VMEM((1, 2 * CH), jnp.int32),
            pltpu.VMEM((1, CH), jnp.int32),
            pltpu.VMEM((1, CH), jnp.int32),
            pltpu.VMEM((2 * CH,), jnp.int32),
            pltpu.VMEM((2 * CH,), jnp.int32),
            pltpu.VMEM((2 * CH, H), jnp.float32),
            pltpu.VMEM((2 * CH, H), jnp.float32),
            pltpu.VMEM((CH, H), jnp.float32),
            pltpu.VMEM((CH, H), jnp.float32),
            pltpu.VMEM_SHARED((NP, H), jnp.float32),
            pltpu.SemaphoreType.DMA,
            pltpu.SemaphoreType.DMA,
            pltpu.SemaphoreType.DMA,
        ],
    )(_sc_body)
    return kern(sd_p, table, phh)


def _out_body(af_ref, aggl_ref, aggu_ref, wl_ref, wu_ref, o_ref):
    o_ref[...] = (af_ref[...]
                  + jnp.dot(aggl_ref[...], wl_ref[...], preferred_element_type=jnp.float32)
                  + jnp.dot(aggu_ref[...], wu_ref[...], preferred_element_type=jnp.float32))


def _outproj(af, aggl, aggu, w_out):
    BA = 2000
    return pl.pallas_call(
        _out_body,
        grid=(N // BA,),
        in_specs=[
            pl.BlockSpec((BA, D), lambda i: (i, 0)),
            pl.BlockSpec((BA, H), lambda i: (i, 0)),
            pl.BlockSpec((BA, H), lambda i: (i, 0)),
            pl.BlockSpec((H, D), lambda i: (0, 0)),
            pl.BlockSpec((H, D), lambda i: (0, 0)),
        ],
        out_specs=pl.BlockSpec((BA, D), lambda i: (i, 0)),
        out_shape=jax.ShapeDtypeStruct((N, D), jnp.float32),
    )(af, aggl, aggu, w_out[:H], w_out[H:])


def kernel(atomic_numbers, edge_index, bond_i, bond_j, embedding, W_angle, W_msg, b_msg, W_out):
    an2d = atomic_numbers.astype(jnp.int32).reshape(N, 1)
    w12 = W_msg[:2 * D]
    w3 = W_msg[2 * D:]

    table, af = _prep(an2d, embedding, w12)

    pad = EP - E
    bi_t = jnp.pad(bond_i, ((0, pad), (0, 0))).T.reshape(3, EP // 128, 128)
    bj_t = jnp.pad(bond_j, ((0, pad), (0, 0))).T.reshape(3, EP // 128, 128)
    phh = _angle(bi_t, bj_t, W_angle, w3, b_msg.reshape(2, 1, H))

    src_c = jnp.pad(edge_index[0].astype(jnp.int32), (0, pad)).reshape(EP // CH, CH)
    dst_c = jnp.pad(edge_index[1].astype(jnp.int32), (0, pad),
                    constant_values=N).reshape(EP // CH, CH)
    sd_p = jnp.concatenate([src_c, dst_c], axis=1).reshape(-1)
    aggp = _sc_aggregate(sd_p, table, phh)

    aggl = aggp[:N]
    aggu = aggp[NP:NP + N]
    return _outproj(af, aggl, aggu, W_out)
